# baseline (speedup 1.0000x reference)
import jax
import jax.numpy as jnp
from jax import lax
from jax.experimental import pallas as pl
from jax.experimental.pallas import tpu as pltpu
from jax.experimental.pallas import tpu_sc as plsc

_NC = 2
_NS = 16
_NW = _NC * _NS
_V = 1000000
_D = 32
_B = 4096
_H = 200
_FULL_TC = _V // 128
_TAIL_V = _FULL_TC * 128
_BLKS_PER_W = 245


def _transpose_body(tabT_hbm, scratch_hbm, blk_v, tbk_v, sem):
    c = lax.axis_index("c")
    s = lax.axis_index("s")
    wid = s * _NC + c

    iota = lax.iota(jnp.int32, 16)

    def block(j, _):
        cblk = j * _NW + wid

        @pl.when(cblk < _FULL_TC)
        def _():
            pltpu.sync_copy(tabT_hbm.at[:, pl.ds(cblk * 128, 128)], blk_v)

            def row(r, _):
                for k in range(8):
                    d0 = 16 * (k % 2)
                    vv = r * 4 + k // 2
                    x = plsc.load_gather(
                        blk_v, [iota + d0, jnp.zeros((16,), jnp.int32) + vv])
                    tbk_v[r, pl.ds(k * 16, 16)] = x
                return ()

            lax.fori_loop(0, 32, row, (), unroll=False)
            pltpu.sync_copy(tbk_v, scratch_hbm.at[pl.ds(cblk * 32, 32)])

        return ()

    lax.fori_loop(0, _BLKS_PER_W, block, (), unroll=False)


def _gather_body(idx_hbm, table_hbm, out_hbm, idx_v, rows0_v, rows1_v,
                 tbk_v, gsem):
    c = lax.axis_index("c")
    s = lax.axis_index("s")
    wid = s * _NC + c

    pltpu.sync_copy(idx_hbm.at[wid], idx_v)
    iota = lax.iota(jnp.int32, 16)
    drain_src = table_hbm.at[pl.ds(0, 128)]

    def fire(h, rows_v):
        pltpu.async_copy(table_hbm.at[idx_v.at[h]], rows_v, gsem)

    def consume(h, rows_v):
        pltpu.make_async_copy(drain_src, rows_v, gsem).wait()

        def drow(d, _):
            for m in range(8):
                x = plsc.load_gather(
                    rows_v, [iota + m * 16, jnp.zeros((16,), jnp.int32) + d])
                tbk_v[d, pl.ds(m * 16, 16)] = x
            return ()

        lax.fori_loop(0, 32, drow, (), unroll=False)
        for tr in range(4):
            pltpu.sync_copy(tbk_v.at[pl.ds(tr * 8, 8)], out_hbm.at[h, tr, wid])

    fire(0, rows0_v)

    def pair(g, _):
        h0 = g * 2
        fire(h0 + 1, rows1_v)
        consume(h0, rows0_v)

        @pl.when(h0 + 2 < _H)
        def _():
            fire(h0 + 2, rows0_v)

        consume(h0 + 1, rows1_v)
        return ()

    lax.fori_loop(0, _H // 2, pair, (), unroll=False)


@jax.jit
def _emb(x, table):
    mesh = plsc.VectorSubcoreMesh(core_axis_name="c", subcore_axis_name="s",
                                  num_cores=_NC, num_subcores=_NS)

    f_tr = pl.kernel(
        _transpose_body,
        out_type=jax.ShapeDtypeStruct((_V // 4, 128), jnp.float32),
        mesh=mesh,
        scratch_types=[
            pltpu.VMEM((32, 128), jnp.float32),
            pltpu.VMEM((32, 128), jnp.float32),
            pltpu.SemaphoreType.DMA,
        ],
        compiler_params=pltpu.CompilerParams(needs_layout_passes=False),
    )
    scratchA = f_tr(table.T)
    tail = table[_TAIL_V:].reshape(16, 128)
    scratch = lax.dynamic_update_slice(scratchA, tail, (_TAIL_V * _D // 128, 0))
    scratch = scratch.reshape(_V, _D)

    idx3 = x.reshape(_NW, _B // _NW, _H).transpose(0, 2, 1)

    f_g = pl.kernel(
        _gather_body,
        out_type=jax.ShapeDtypeStruct((_H, 4, _NW, 8, 128), jnp.float32),
        mesh=mesh,
        scratch_types=[
            pltpu.VMEM((_H, 128), jnp.int32),
            pltpu.VMEM((128, _D), jnp.float32),
            pltpu.VMEM((128, _D), jnp.float32),
            pltpu.VMEM((_D, 128), jnp.float32),
            pltpu.SemaphoreType.DMA,
        ],
        compiler_params=pltpu.CompilerParams(use_tc_tiling_on_sc=False,
                                             needs_layout_passes=False),
    )
    out5 = f_g(idx3, scratch)
    return out5.transpose(2, 4, 0, 1, 3).reshape(_B, _H, _D)


def kernel(x, table):
    return _emb(x.astype(jnp.int32), table)


# speedup vs baseline: 1.2367x; 1.2367x over previous
import jax
import jax.numpy as jnp
from jax import lax
from jax.experimental import pallas as pl
from jax.experimental.pallas import tpu as pltpu
from jax.experimental.pallas import tpu_sc as plsc

_NC = 2
_NS = 16
_NW = _NC * _NS
_V = 1000000
_D = 32
_B = 4096
_H = 200
_FULL_TC = _V // 128
_TAIL_V = _FULL_TC * 128
_NPAIR = 123


def _transpose_body(tabT_hbm, scratch_hbm,
                    blk0, blk1, tbk0, tbk1, isem0, isem1, osem0, osem1):
    c = lax.axis_index("c")
    s = lax.axis_index("s")
    wid = s * _NC + c
    iota = lax.iota(jnp.int32, 16)

    def fire_in(j, blk, isem):
        cb = j * _NW + wid

        @pl.when(cb < _FULL_TC)
        def _():
            pltpu.async_copy(tabT_hbm.at[:, pl.ds(cb * 128, 128)], blk, isem)

    def transpose(blk, tbk):
        def row4(r4, _):
            for rr in range(4):
                r = r4 * 4 + rr
                for k in range(8):
                    d0 = 16 * (k % 2)
                    x = plsc.load_gather(
                        blk, [iota + d0,
                              jnp.zeros((16,), jnp.int32) + (r * 4 + k // 2)])
                    tbk[r, pl.ds(k * 16, 16)] = x
            return ()

        lax.fori_loop(0, 8, row4, (), unroll=False)

    def step(j, blk, tbk, isem, osem):
        cb = j * _NW + wid
        cbp = (j - 2) * _NW + wid

        @pl.when(cb < _FULL_TC)
        def _():
            pltpu.make_async_copy(
                tabT_hbm.at[:, pl.ds(0, 128)], blk, isem).wait()

        @pl.when((j >= 2) & (cbp < _FULL_TC))
        def _():
            pltpu.make_async_copy(
                tbk, scratch_hbm.at[pl.ds(0, 32)], osem).wait()

        @pl.when(cb < _FULL_TC)
        def _():
            transpose(blk, tbk)
            pltpu.async_copy(tbk, scratch_hbm.at[pl.ds(cb * 32, 32)], osem)

        fire_in(j + 2, blk, isem)

    fire_in(0, blk0, isem0)
    fire_in(1, blk1, isem1)

    def pair(p, _):
        step(p * 2, blk0, tbk0, isem0, osem0)
        step(p * 2 + 1, blk1, tbk1, isem1, osem1)
        return ()

    lax.fori_loop(0, _NPAIR, pair, (), unroll=False)

    @pl.when(244 * _NW + wid < _FULL_TC)
    def _():
        pltpu.make_async_copy(tbk0, scratch_hbm.at[pl.ds(0, 32)], osem0).wait()


def _gather_body(idx_hbm, table_hbm, out_hbm, idx_v,
                 rows0, rows1, rows2, rows3, tbk0, tbk1,
                 gsem0, gsem1, gsem2, gsem3, wsem0, wsem1):
    c = lax.axis_index("c")
    s = lax.axis_index("s")
    wid = s * _NC + c
    iota = lax.iota(jnp.int32, 16)

    pltpu.sync_copy(idx_hbm.at[wid], idx_v)
    rows = (rows0, rows1, rows2, rows3)
    gsems = (gsem0, gsem1, gsem2, gsem3)
    tbks = (tbk0, tbk1)
    wsems = (wsem0, wsem1)

    def fire(h, rv, gs):
        pltpu.async_copy(table_hbm.at[idx_v.at[h]], rv, gs)

    for b in range(4):
        fire(b, rows[b], gsems[b])

    def quad(p, _):
        for b in range(4):
            h = p * 4 + b
            rv, gs = rows[b], gsems[b]
            tb, ws = tbks[b % 2], wsems[b % 2]
            pltpu.make_async_copy(table_hbm.at[pl.ds(0, 128)], rv, gs).wait()
            if b >= 2:
                pltpu.make_async_copy(
                    tb, out_hbm.at[0, :, 0], ws).wait()
            else:
                @pl.when(p > 0)
                def _():
                    pltpu.make_async_copy(
                        tb, out_hbm.at[0, :, 0], ws).wait()

            def d8(dd, _):
                for kk in range(4):
                    d = dd * 4 + kk
                    tr = d // 8
                    sl = d % 8
                    for m in range(8):
                        x = plsc.load_gather(
                            rv, [iota + m * 16,
                                 jnp.zeros((16,), jnp.int32) + d])
                        tb[tr, sl, pl.ds(m * 16, 16)] = x
                return ()

            lax.fori_loop(0, 8, d8, (), unroll=False)
            pltpu.async_copy(tb, out_hbm.at[h, :, wid], ws)

            @pl.when(h + 4 < _H)
            def _():
                fire(h + 4, rv, gs)
        return ()

    lax.fori_loop(0, _H // 4, quad, (), unroll=False)
    pltpu.make_async_copy(tbk0, out_hbm.at[0, :, 0], wsem0).wait()
    pltpu.make_async_copy(tbk1, out_hbm.at[0, :, 0], wsem1).wait()


@jax.jit
def _emb(x, table):
    mesh = plsc.VectorSubcoreMesh(core_axis_name="c", subcore_axis_name="s",
                                  num_cores=_NC, num_subcores=_NS)

    f_tr = pl.kernel(
        _transpose_body,
        out_type=jax.ShapeDtypeStruct((_V // 4, 128), jnp.float32),
        mesh=mesh,
        scratch_types=[
            pltpu.VMEM((32, 128), jnp.float32),
            pltpu.VMEM((32, 128), jnp.float32),
            pltpu.VMEM((32, 128), jnp.float32),
            pltpu.VMEM((32, 128), jnp.float32),
            pltpu.SemaphoreType.DMA,
            pltpu.SemaphoreType.DMA,
            pltpu.SemaphoreType.DMA,
            pltpu.SemaphoreType.DMA,
        ],
        compiler_params=pltpu.CompilerParams(needs_layout_passes=False),
    )
    scratchA = f_tr(table.T)
    tail = table[_TAIL_V:].reshape(16, 128)
    scratch = lax.dynamic_update_slice(scratchA, tail, (_TAIL_V * _D // 128, 0))
    scratch = scratch.reshape(_V, _D)

    idx3 = x.reshape(_NW, _B // _NW, _H).transpose(0, 2, 1)

    f_g = pl.kernel(
        _gather_body,
        out_type=jax.ShapeDtypeStruct((_H, 4, _NW, 8, 128), jnp.float32),
        mesh=mesh,
        scratch_types=[
            pltpu.VMEM((_H, 128), jnp.int32),
            pltpu.VMEM((128, _D), jnp.float32),
            pltpu.VMEM((128, _D), jnp.float32),
            pltpu.VMEM((128, _D), jnp.float32),
            pltpu.VMEM((128, _D), jnp.float32),
            pltpu.VMEM((4, 8, 128), jnp.float32),
            pltpu.VMEM((4, 8, 128), jnp.float32),
            pltpu.SemaphoreType.DMA,
            pltpu.SemaphoreType.DMA,
            pltpu.SemaphoreType.DMA,
            pltpu.SemaphoreType.DMA,
            pltpu.SemaphoreType.DMA,
            pltpu.SemaphoreType.DMA,
        ],
        compiler_params=pltpu.CompilerParams(use_tc_tiling_on_sc=False,
                                             needs_layout_passes=False),
    )
    out5 = f_g(idx3, scratch)
    return out5.transpose(2, 4, 0, 1, 3).reshape(_B, _H, _D)


def kernel(x, table):
    return _emb(x.astype(jnp.int32), table)


# speedup vs baseline: 1.6913x; 1.3677x over previous
import jax
import jax.numpy as jnp
from jax import lax
from jax.experimental import pallas as pl
from jax.experimental.pallas import tpu as pltpu
from jax.experimental.pallas import tpu_sc as plsc

_NC = 2
_NS = 16
_NW = _NC * _NS
_V = 1000000
_D = 32
_B = 4096
_H = 200
_FULL_TC = _V // 128
_TAIL_V = _FULL_TC * 128
_NG = _FULL_TC // 4


def _transpose_body(tabT_hbm, scratch_hbm,
                    blk0, blk1, tbk0, tbk1, isem0, isem1, osem0, osem1):
    c = lax.axis_index("c")
    s = lax.axis_index("s")
    wid = s * _NC + c
    iota = lax.iota(jnp.int32, 16)
    iotas = (iota, iota + 16)

    def fire_in(j, blk, isem):
        g = j * _NW + wid

        @pl.when(g < _NG)
        def _():
            pltpu.async_copy(tabT_hbm.at[:, pl.ds(g * 512, 512)], blk, isem)

    def transpose(blk, tbk):
        def row(r, _):
            xs = []
            for q in range(4):
                vs = [jnp.zeros((16,), jnp.int32) + (q * 128 + r * 4 + qq)
                      for qq in range(4)]
                for k in range(8):
                    xs.append(plsc.load_gather(blk, [iotas[k % 2], vs[k // 2]]))
            for q in range(4):
                for k in range(8):
                    tbk[q * 32 + r, pl.ds(k * 16, 16)] = xs[q * 8 + k]
            return ()

        lax.fori_loop(0, 32, row, (), unroll=False)

    def step(j, blk, tbk, isem, osem):
        g = j * _NW + wid
        gp = (j - 2) * _NW + wid

        @pl.when(g < _NG)
        def _():
            pltpu.make_async_copy(
                tabT_hbm.at[:, pl.ds(0, 512)], blk, isem).wait()

        @pl.when((j >= 2) & (gp < _NG))
        def _():
            pltpu.make_async_copy(
                tbk, scratch_hbm.at[pl.ds(0, 128)], osem).wait()

        @pl.when(g < _NG)
        def _():
            transpose(blk, tbk)
            pltpu.async_copy(tbk, scratch_hbm.at[pl.ds(g * 128, 128)], osem)

        fire_in(j + 2, blk, isem)

    fire_in(0, blk0, isem0)
    fire_in(1, blk1, isem1)

    def pair(p, _):
        step(p * 2, blk0, tbk0, isem0, osem0)
        step(p * 2 + 1, blk1, tbk1, isem1, osem1)
        return ()

    lax.fori_loop(0, 31, pair, (), unroll=False)

    @pl.when(60 * _NW + wid < _NG)
    def _():
        pltpu.make_async_copy(tbk0, scratch_hbm.at[pl.ds(0, 128)], osem0).wait()

    @pl.when(61 * _NW + wid < _NG)
    def _():
        pltpu.make_async_copy(tbk1, scratch_hbm.at[pl.ds(0, 128)], osem1).wait()


def _gather_body(idx_hbm, table_hbm, out_hbm, idx_v,
                 rows0, rows1, rows2, rows3, tbk0, tbk1,
                 gsem0, gsem1, gsem2, gsem3, wsem0, wsem1):
    c = lax.axis_index("c")
    s = lax.axis_index("s")
    wid = s * _NC + c
    iota = lax.iota(jnp.int32, 16)

    pltpu.sync_copy(idx_hbm.at[wid], idx_v)
    rows = (rows0, rows1, rows2, rows3)
    gsems = (gsem0, gsem1, gsem2, gsem3)
    tbks = (tbk0, tbk1)
    wsems = (wsem0, wsem1)

    def fire(t, rv, gs):
        pltpu.async_copy(table_hbm.at[idx_v.at[2 * t]],
                         rv.at[pl.ds(0, 128)], gs)
        pltpu.async_copy(table_hbm.at[idx_v.at[2 * t + 1]],
                         rv.at[pl.ds(128, 128)], gs)

    for b in range(4):
        fire(b, rows[b], gsems[b])

    def quad(p, _):
        for b in range(4):
            t = p * 4 + b
            h0 = 2 * t
            rv, gs = rows[b], gsems[b]
            tb, ws = tbks[b % 2], wsems[b % 2]
            pltpu.make_async_copy(table_hbm.at[pl.ds(0, 256)], rv, gs).wait()
            if b >= 2:
                pltpu.make_async_copy(tb, out_hbm.at[pl.ds(0, 2), :, 0],
                                      ws).wait()
            else:
                @pl.when(p > 0)
                def _():
                    pltpu.make_async_copy(tb, out_hbm.at[pl.ds(0, 2), :, 0],
                                          ws).wait()

            def d8(dd, _):
                for u in range(2):
                    xs = []
                    for kk in range(4):
                        d = dd * 4 + kk
                        dsplat = jnp.zeros((16,), jnp.int32) + d
                        for m in range(8):
                            xs.append(plsc.load_gather(
                                rv, [iota + (u * 128 + m * 16), dsplat]))
                    for kk in range(4):
                        d = dd * 4 + kk
                        for m in range(8):
                            tb[u, d // 8, d % 8, pl.ds(m * 16, 16)] = \
                                xs[kk * 8 + m]
                return ()

            lax.fori_loop(0, 8, d8, (), unroll=False)
            pltpu.async_copy(tb, out_hbm.at[pl.ds(h0, 2), :, wid], ws)

            @pl.when(t + 4 < _H // 2)
            def _():
                fire(t + 4, rv, gs)
        return ()

    lax.fori_loop(0, _H // 8, quad, (), unroll=False)
    pltpu.make_async_copy(tbk0, out_hbm.at[pl.ds(0, 2), :, 0], wsem0).wait()
    pltpu.make_async_copy(tbk1, out_hbm.at[pl.ds(0, 2), :, 0], wsem1).wait()


@jax.jit
def _emb(x, table):
    mesh = plsc.VectorSubcoreMesh(core_axis_name="c", subcore_axis_name="s",
                                  num_cores=_NC, num_subcores=_NS)

    f_tr = pl.kernel(
        _transpose_body,
        out_type=jax.ShapeDtypeStruct((_V // 4, 128), jnp.float32),
        mesh=mesh,
        scratch_types=[
            pltpu.VMEM((32, 512), jnp.float32),
            pltpu.VMEM((32, 512), jnp.float32),
            pltpu.VMEM((128, 128), jnp.float32),
            pltpu.VMEM((128, 128), jnp.float32),
            pltpu.SemaphoreType.DMA,
            pltpu.SemaphoreType.DMA,
            pltpu.SemaphoreType.DMA,
            pltpu.SemaphoreType.DMA,
        ],
        compiler_params=pltpu.CompilerParams(needs_layout_passes=False,
                                             disable_bounds_checks=True),
    )
    scratchA = f_tr(table.T)
    tail = table[_TAIL_V:].reshape(16, 128)
    scratch = lax.dynamic_update_slice(scratchA, tail, (_TAIL_V * _D // 128, 0))
    scratch = scratch.reshape(_V, _D)

    idx3 = x.reshape(_NW, _B // _NW, _H).transpose(0, 2, 1)

    f_g = pl.kernel(
        _gather_body,
        out_type=jax.ShapeDtypeStruct((_H, 4, _NW, 8, 128), jnp.float32),
        mesh=mesh,
        scratch_types=[
            pltpu.VMEM((_H, 128), jnp.int32),
            pltpu.VMEM((256, _D), jnp.float32),
            pltpu.VMEM((256, _D), jnp.float32),
            pltpu.VMEM((256, _D), jnp.float32),
            pltpu.VMEM((256, _D), jnp.float32),
            pltpu.VMEM((2, 4, 8, 128), jnp.float32),
            pltpu.VMEM((2, 4, 8, 128), jnp.float32),
            pltpu.SemaphoreType.DMA,
            pltpu.SemaphoreType.DMA,
            pltpu.SemaphoreType.DMA,
            pltpu.SemaphoreType.DMA,
            pltpu.SemaphoreType.DMA,
            pltpu.SemaphoreType.DMA,
        ],
        compiler_params=pltpu.CompilerParams(use_tc_tiling_on_sc=False,
                                             needs_layout_passes=False,
                                             disable_bounds_checks=True),
    )
    out5 = f_g(idx3, scratch)
    return out5.transpose(2, 4, 0, 1, 3).reshape(_B, _H, _D)


def kernel(x, table):
    return _emb(x.astype(jnp.int32), table)


# speedup vs baseline: 1.7343x; 1.0254x over previous
import jax
import jax.numpy as jnp
from jax import lax
from jax.experimental import pallas as pl
from jax.experimental.pallas import tpu as pltpu
from jax.experimental.pallas import tpu_sc as plsc

_NC = 2
_NS = 16
_NW = _NC * _NS
_V = 1000000
_D = 32
_B = 4096
_H = 200
_FULL_TC = _V // 128
_TAIL_V = _FULL_TC * 128
_NG = _FULL_TC // 4


def _transpose_body(tabT_hbm, scratch_hbm,
                    blk0, blk1, tbk0, tbk1, isem0, isem1, osem0, osem1):
    c = lax.axis_index("c")
    s = lax.axis_index("s")
    wid = s * _NC + c
    iota = lax.iota(jnp.int32, 16)
    iotas = (iota, iota + 16)

    def fire_in(j, blk, isem):
        g = j * _NW + wid

        @pl.when(g < _NG)
        def _():
            pltpu.async_copy(tabT_hbm.at[:, pl.ds(g * 512, 512)], blk, isem)

    iota32 = iota * 32

    def transpose(blk, tbk):
        def vchunk(vc, _):
            v0 = vc * 16
            cols = jnp.zeros((16,), jnp.int32) + v0 + iota
            base = jnp.zeros((16,), jnp.int32) + (v0 * 32) + iota32
            xs = []
            dvs = []
            for s in range(0, 32, 1):
                dv = (iota + s) & 31
                dvs.append(dv)
                xs.append(plsc.load_gather(blk, [dv, cols]))
                if s % 8 == 7:
                    for q in range(8):
                        plsc.store_scatter(tbk, [base + dvs[q]], xs[q])
                    xs = []
                    dvs = []
            return ()

        lax.fori_loop(0, 32, vchunk, (), unroll=False)

    def step(j, blk, tbk, isem, osem):
        g = j * _NW + wid
        gp = (j - 2) * _NW + wid

        @pl.when(g < _NG)
        def _():
            pltpu.make_async_copy(
                tabT_hbm.at[:, pl.ds(0, 512)], blk, isem).wait()

        @pl.when((j >= 2) & (gp < _NG))
        def _():
            pltpu.make_async_copy(
                tbk, scratch_hbm.at[pl.ds(0, 16384)], osem).wait()

        @pl.when(g < _NG)
        def _():
            transpose(blk, tbk)
            pltpu.async_copy(tbk, scratch_hbm.at[pl.ds(g * 16384, 16384)], osem)

        fire_in(j + 2, blk, isem)

    fire_in(0, blk0, isem0)
    fire_in(1, blk1, isem1)

    def pair(p, _):
        step(p * 2, blk0, tbk0, isem0, osem0)
        step(p * 2 + 1, blk1, tbk1, isem1, osem1)
        return ()

    lax.fori_loop(0, 31, pair, (), unroll=False)

    @pl.when(60 * _NW + wid < _NG)
    def _():
        pltpu.make_async_copy(tbk0, scratch_hbm.at[pl.ds(0, 16384)], osem0).wait()

    @pl.when(61 * _NW + wid < _NG)
    def _():
        pltpu.make_async_copy(tbk1, scratch_hbm.at[pl.ds(0, 16384)], osem1).wait()


def _gather_body(idx_hbm, table_hbm, out_hbm, idx_v,
                 rows0, rows1, rows2, rows3, tbk0, tbk1,
                 gsem0, gsem1, gsem2, gsem3, wsem0, wsem1):
    c = lax.axis_index("c")
    s = lax.axis_index("s")
    wid = s * _NC + c
    iota = lax.iota(jnp.int32, 16)

    pltpu.sync_copy(idx_hbm.at[wid], idx_v)
    rows = (rows0, rows1, rows2, rows3)
    gsems = (gsem0, gsem1, gsem2, gsem3)
    tbks = (tbk0, tbk1)
    wsems = (wsem0, wsem1)

    def fire(t, rv, gs):
        pltpu.async_copy(table_hbm.at[idx_v.at[2 * t]],
                         rv.at[pl.ds(0, 128)], gs)
        pltpu.async_copy(table_hbm.at[idx_v.at[2 * t + 1]],
                         rv.at[pl.ds(128, 128)], gs)

    for b in range(4):
        fire(b, rows[b], gsems[b])

    def quad(p, _):
        for b in range(4):
            t = p * 4 + b
            h0 = 2 * t
            rv, gs = rows[b], gsems[b]
            tb, ws = tbks[b % 2], wsems[b % 2]
            pltpu.make_async_copy(table_hbm.at[pl.ds(0, 256)], rv, gs).wait()
            if b >= 2:
                pltpu.make_async_copy(tb.at[:, :, :, pl.ds(0, 128)],
                                      out_hbm.at[pl.ds(0, 2), :, 0],
                                      ws).wait()
            else:
                @pl.when(p > 0)
                def _():
                    pltpu.make_async_copy(tb.at[:, :, :, pl.ds(0, 128)],
                                          out_hbm.at[pl.ds(0, 2), :, 0],
                                          ws).wait()

            def bl4(bb, _):
                for bj in range(4):
                    bl = bb * 4 + bj
                    blsplat = jnp.zeros((16,), jnp.int32) + bl
                    for u in range(2):
                        row = u * 128 + bl
                        for d0 in (0, 16):
                            x = rv[row, pl.ds(d0, 16)]
                            dc = u * 32 + d0 + iota
                            plsc.store_scatter(
                                tb, [dc >> 5, (dc >> 3) & 3, dc & 7, blsplat],
                                x)
                return ()

            lax.fori_loop(0, 32, bl4, (), unroll=False)
            pltpu.async_copy(tb.at[:, :, :, pl.ds(0, 128)],
                             out_hbm.at[pl.ds(h0, 2), :, wid], ws)

            @pl.when(t + 4 < _H // 2)
            def _():
                fire(t + 4, rv, gs)
        return ()

    lax.fori_loop(0, _H // 8, quad, (), unroll=False)
    pltpu.make_async_copy(tbk0.at[:, :, :, pl.ds(0, 128)],
                          out_hbm.at[pl.ds(0, 2), :, 0], wsem0).wait()
    pltpu.make_async_copy(tbk1.at[:, :, :, pl.ds(0, 128)],
                          out_hbm.at[pl.ds(0, 2), :, 0], wsem1).wait()


@jax.jit
def _emb(x, table):
    mesh = plsc.VectorSubcoreMesh(core_axis_name="c", subcore_axis_name="s",
                                  num_cores=_NC, num_subcores=_NS)

    f_tr = pl.kernel(
        _transpose_body,
        out_type=jax.ShapeDtypeStruct((_V * _D,), jnp.float32),
        mesh=mesh,
        scratch_types=[
            pltpu.VMEM((32, 512), jnp.float32),
            pltpu.VMEM((32, 512), jnp.float32),
            pltpu.VMEM((16384,), jnp.float32),
            pltpu.VMEM((16384,), jnp.float32),
            pltpu.SemaphoreType.DMA,
            pltpu.SemaphoreType.DMA,
            pltpu.SemaphoreType.DMA,
            pltpu.SemaphoreType.DMA,
        ],
        compiler_params=pltpu.CompilerParams(needs_layout_passes=False,
                                             disable_bounds_checks=True),
    )
    scratchA = f_tr(table.T).reshape(_V, _D)
    scratch = lax.dynamic_update_slice(scratchA, table[_TAIL_V:], (_TAIL_V, 0))

    idx3 = x.reshape(_NW, _B // _NW, _H).transpose(0, 2, 1)

    f_g = pl.kernel(
        _gather_body,
        out_type=jax.ShapeDtypeStruct((_H, 4, _NW, 8, 128), jnp.float32),
        mesh=mesh,
        scratch_types=[
            pltpu.VMEM((_H, 128), jnp.int32),
            pltpu.VMEM((256, _D), jnp.float32),
            pltpu.VMEM((256, _D), jnp.float32),
            pltpu.VMEM((256, _D), jnp.float32),
            pltpu.VMEM((256, _D), jnp.float32),
            pltpu.VMEM((2, 4, 8, 129), jnp.float32),
            pltpu.VMEM((2, 4, 8, 129), jnp.float32),
            pltpu.SemaphoreType.DMA,
            pltpu.SemaphoreType.DMA,
            pltpu.SemaphoreType.DMA,
            pltpu.SemaphoreType.DMA,
            pltpu.SemaphoreType.DMA,
            pltpu.SemaphoreType.DMA,
        ],
        compiler_params=pltpu.CompilerParams(use_tc_tiling_on_sc=False,
                                             needs_layout_passes=False,
                                             disable_bounds_checks=True),
    )
    out5 = f_g(idx3, scratch)
    return out5.transpose(2, 4, 0, 1, 3).reshape(_B, _H, _D)


def kernel(x, table):
    return _emb(x.astype(jnp.int32), table)
